# baseline (speedup 1.0000x reference)
import functools

import jax
import jax.numpy as jnp
from jax.experimental import pallas as pl
from jax.experimental.pallas import tpu as pltpu
from jax.experimental.pallas import tpu_sc as plsc

K = 8192
D = 32
N = 8192
RB = 256
KT = 2048
NB = N // RB
NKT = K // KT


GW = 128 // D


def _sc_gather(emb_grp, gidx):
    info = plsc.get_sparse_core_info()
    nw = info.num_cores * info.num_subcores
    bpw = N // nw
    mesh = plsc.VectorSubcoreMesh(core_axis_name="c", subcore_axis_name="s")

    @functools.partial(
        pl.kernel, mesh=mesh,
        out_type=jax.ShapeDtypeStruct((N, 128), jnp.float32),
        scratch_types=[
            pltpu.VMEM((bpw,), jnp.int32),
            pltpu.VMEM((bpw, 128), jnp.float32),
            pltpu.SemaphoreType.DMA,
        ],
    )
    def k(table_hbm, idx_hbm, out_hbm, idx_v, rows_v, sem):
        wid = jax.lax.axis_index("s") * info.num_cores + jax.lax.axis_index("c")
        base = wid * bpw
        pltpu.sync_copy(idx_hbm.at[pl.ds(base, bpw)], idx_v)
        pltpu.async_copy(table_hbm.at[idx_v], rows_v, sem).wait()
        pltpu.sync_copy(rows_v, out_hbm.at[pl.ds(base, bpw)])

    return k(emb_grp, gidx)


def _stats_kernel(zf_ref, zqr_ref, tok_ref, zq_ref, sc_ref, hist_ref, acc_ref):
    i = pl.program_id(0)

    @pl.when(i == 0)
    def _init():
        hist_ref[...] = jnp.zeros_like(hist_ref)
        acc_ref[0] = 0.0

    zf = zf_ref[...]
    grp = zqr_ref[...]
    mini = tok_ref[0, 0, :]
    sub = (mini & (GW - 1))[:, None]
    zq = grp[:, 0:D]
    for s in range(1, GW):
        zq = jnp.where(sub == s, grp[:, s * D:(s + 1) * D], zq)

    for j in range(NKT):
        ids = jax.lax.broadcasted_iota(jnp.int32, (RB, KT), 1) + j * KT
        ohf = (mini[:, None] == ids).astype(jnp.float32)
        hist_ref[j, :] += jnp.sum(ohf, axis=0)

    zq_ref[...] = zf + (zq - zf)
    acc_ref[0] += jnp.sum((zq - zf) * (zq - zf))

    @pl.when(i == NB - 1)
    def _fin():
        hist = hist_ref[...]
        total = acc_ref[0]
        loss = 1.25 * total / (N * D)
        qerr = total / N
        p = hist / jnp.sum(hist)
        perp = jnp.exp(-jnp.sum(p * jnp.log(p + 1e-10)))
        util = jnp.sum((hist > 0).astype(jnp.float32)) / K
        sc_ref[0, :] = jnp.stack([loss, qerr, util, perp])


def _stats(zf, zq_rows, token):
    return pl.pallas_call(
        _stats_kernel,
        grid=(NB,),
        in_specs=[
            pl.BlockSpec((RB, D), lambda i: (i, 0)),
            pl.BlockSpec((RB, 128), lambda i: (i, 0)),
            pl.BlockSpec((1, 1, RB), lambda i: (i, 0, 0)),
        ],
        out_specs=[
            pl.BlockSpec((RB, D), lambda i: (i, 0)),
            pl.BlockSpec((1, 4), lambda i: (0, 0)),
        ],
        out_shape=[
            jax.ShapeDtypeStruct((N, D), jnp.float32),
            jax.ShapeDtypeStruct((1, 4), jnp.float32),
        ],
        scratch_shapes=[
            pltpu.VMEM((NKT, KT), jnp.float32),
            pltpu.SMEM((1,), jnp.float32),
        ],
    )(zf, zq_rows, token)


def kernel(z, emb_w, embed_prob):
    del embed_prob
    zp = jnp.transpose(z, (0, 2, 3, 1))
    zf = zp.reshape(-1, D)
    dist = (jnp.sum(zf ** 2, axis=1, keepdims=True) + jnp.sum(emb_w ** 2, axis=1)
            - 2.0 * (zf @ emb_w.T))
    token = jnp.argmin(dist, axis=1).astype(jnp.int32)
    zq_rows = _sc_gather(emb_w.reshape(K // GW, 128), token >> 2)
    zq, scal = _stats(zf, zq_rows, token.reshape(NB, 1, RB))
    z_q_out = jnp.transpose(zq.reshape(zp.shape), (0, 3, 1, 2))
    return (z_q_out, scal[0, 0], scal[0, 1], scal[0, 2], scal[0, 3])
